# baseline (speedup 1.0000x reference)
import functools

import jax
import jax.numpy as jnp
import numpy as np
from jax import lax
from jax.experimental import pallas as pl
from jax.experimental.pallas import tpu as pltpu
from jax.experimental.pallas import tpu_sc as plsc

_N = 4096
_E = 32768
_M = 16

_NC = 2
_NS = 16
_NW = _NC * _NS

_EDGE_BLOCK = 512
_NODE_BLOCK = 1024


def _edge_body(feat_ref, hs_ref, bb_ref,
               w1_ref, b1_ref, g1_ref, be1_ref,
               w2_ref, b2_ref, g2_ref, be2_ref, mg_ref, w3_ref, b3_ref,
               t16_ref, s256_ref, sel_ref, s224_ref, out_ref):
    f32 = jnp.float32

    def mm(a, b):
        return jnp.dot(a, b, preferred_element_type=f32)

    mg = mg_ref[...]

    def ln_relu(y, g, be):
        mu = mm(y, mg)
        var = mm(y * y, mg) - mu * mu
        yn = (y - mu) / jnp.sqrt(var + 1e-5) * g + be
        return jnp.maximum(yn, 0.0)

    y = mm(feat_ref[...], w1_ref[...]) + b1_ref[...]
    y = ln_relu(y, g1_ref[...], be1_ref[...])
    y = mm(y, w2_ref[...]) + b2_ref[...]
    z = ln_relu(y, g2_ref[...], be2_ref[...])
    r = mm(z, w3_ref[...]) + b3_ref[...]

    hs = hs_ref[...]
    t16 = t16_ref[...]
    s256 = s256_ref[...]

    ht = [mm(hs[:, 16 * j:16 * j + 16], t16) for j in range(4)]
    rs = [r[:, 256 * k:256 * k + 256] for k in range(6)]

    prods = [rs[0] * ht[0], rs[1] * ht[0]]
    prods += [rs[2] * ht[1 + mi] for mi in range(3)]
    for f in range(3):
        prods += [rs[3 + f] * ht[1 + mi] for mi in range(3)]
    qall = jnp.concatenate([mm(p, s256) for p in prods], axis=1)

    bt = mm(bb_ref[...], sel_ref[...])
    s224 = s224_ref[...]
    parts = [mm(qall * bt[:, 224 * j:224 * j + 224], s224) for j in range(4)]
    parts.append(jnp.ones((qall.shape[0], 16), f32))
    parts.append(jnp.zeros((qall.shape[0], 48), f32))
    out_ref[...] = jnp.concatenate(parts, axis=1)


def _run_edge(feat, hs, bb, pp, interpret=False):
    bsz = _EDGE_BLOCK
    grid = (_E // bsz,)

    def blk(shape):
        return pl.BlockSpec(shape, lambda i: (i, 0))

    def full(a):
        return pl.BlockSpec(a.shape, lambda i: (0, 0))

    in_specs = [blk((bsz, 32)), blk((bsz, 128)), blk((bsz, 34))]
    in_specs += [full(p) for p in pp]
    return pl.pallas_call(
        _edge_body,
        grid=grid,
        in_specs=in_specs,
        out_specs=blk((bsz, 128)),
        out_shape=jax.ShapeDtypeStruct((_E, 128), jnp.float32),
        interpret=interpret,
    )(feat, hs, bb, *pp)


def _combine_body(p0_ref, p1_ref, hc_ref, s0t_ref, k1_ref, o0_ref, o1_ref):
    f32 = jnp.float32
    acc = p0_ref[...] + p1_ref[...]
    cnt = acc[:, 64:65]
    denom = jnp.maximum(cnt, 1.0)
    hc = hc_ref[...]
    s0 = jnp.dot(hc[:, 0:16], s0t_ref[...], preferred_element_type=f32)
    s1 = jnp.dot(hc[:, 16:64], k1_ref[...], preferred_element_type=f32)
    o0_ref[...] = (acc[:, 0:16] + cnt * s0) / denom
    o1_ref[...] = (acc[:, 16:64] + cnt * s1) / denom


def _run_combine(part, hcat, s0t, k1, interpret=False):
    bsz = _NODE_BLOCK
    grid = (_N // bsz,)

    def blk(shape):
        return pl.BlockSpec(shape, lambda i: (i, 0))

    def full(a):
        return pl.BlockSpec(a.shape, lambda i: (0, 0))

    return pl.pallas_call(
        _combine_body,
        grid=grid,
        in_specs=[blk((bsz, 128)), blk((bsz, 128)), blk((bsz, 64)),
                  full(s0t), full(k1)],
        out_specs=[blk((bsz, 16)), blk((bsz, 48))],
        out_shape=[jax.ShapeDtypeStruct((_N, 16), jnp.float32),
                   jax.ShapeDtypeStruct((_N, 48), jnp.float32)],
        interpret=interpret,
    )(part[:_N], part[_N:], hcat, s0t, k1)


def _sc_mesh():
    return plsc.VectorSubcoreMesh(core_axis_name="c", subcore_axis_name="s",
                                  num_cores=_NC, num_subcores=_NS)


def _sc_gather(table, idx2d):
    rows_per_w = _E // _NW
    chunks = rows_per_w // 128

    @functools.partial(
        pl.kernel,
        out_type=jax.ShapeDtypeStruct((_E, 128), jnp.float32),
        mesh=_sc_mesh(),
        scratch_types=[pltpu.VMEM((chunks, 128), jnp.int32),
                       pltpu.VMEM((128, 128), jnp.float32),
                       pltpu.VMEM((128, 128), jnp.float32),
                       pltpu.SemaphoreType.DMA,
                       pltpu.SemaphoreType.DMA],
    )
    def gk(table_hbm, idx_hbm, out_hbm, idx_v, row0, row1, sem0, sem1):
        wid = lax.axis_index("s") * _NC + lax.axis_index("c")
        base = wid * rows_per_w
        pltpu.sync_copy(idx_hbm.at[pl.ds(wid * chunks, chunks)], idx_v)
        bufs = (row0, row1)
        sems = (sem0, sem1)
        descs = [None, None]
        for j in range(chunks):
            k = j % 2
            if descs[k] is not None:
                descs[k].wait()
                pltpu.sync_copy(bufs[k], out_hbm.at[pl.ds(base + (j - 2) * 128,
                                                          128)])
            descs[k] = pltpu.async_copy(table_hbm.at[idx_v.at[j]], bufs[k],
                                        sems[k])
        for j in range(chunks - 2, chunks):
            k = j % 2
            descs[k].wait()
            pltpu.sync_copy(bufs[k], out_hbm.at[pl.ds(base + j * 128, 128)])

    return gk(table, idx2d)


def _sc_scatter(msg, idx2d, zeros):
    rows_per_w = _E // _NW
    chunks = rows_per_w // 128
    stripe = _N // _NS

    @functools.partial(
        pl.kernel,
        out_type=jax.ShapeDtypeStruct((2 * _N, 128), jnp.float32),
        mesh=_sc_mesh(),
        scratch_types=[pltpu.VMEM((chunks, 128), jnp.int32),
                       pltpu.VMEM((128, 128), jnp.float32),
                       pltpu.VMEM((128, 128), jnp.float32),
                       pltpu.VMEM_SHARED((_N, 128), jnp.float32),
                       pltpu.SemaphoreType.DMA,
                       pltpu.SemaphoreType.DMA],
    )
    def sk(msg_hbm, idx_hbm, z_hbm, out_hbm, idx_v, buf0, buf1, acc_sh,
           sem0, sem1):
        cid = lax.axis_index("c")
        sid = lax.axis_index("s")
        wid = sid * _NC + cid
        base = wid * rows_per_w
        pltpu.sync_copy(z_hbm.at[pl.ds(sid * stripe, stripe)],
                        acc_sh.at[pl.ds(sid * stripe, stripe)])
        plsc.subcore_barrier()
        pltpu.sync_copy(idx_hbm.at[pl.ds(wid * chunks, chunks)], idx_v)
        bufs = (buf0, buf1)
        sems = (sem0, sem1)
        descs = [None, None]
        for j in range(chunks):
            k = j % 2
            if descs[k] is not None:
                descs[k].wait()
                pltpu.sync_copy(bufs[k], acc_sh.at[idx_v.at[j - 2]], add=True)
            descs[k] = pltpu.async_copy(
                msg_hbm.at[pl.ds(base + j * 128, 128)], bufs[k], sems[k])
        for j in range(chunks - 2, chunks):
            k = j % 2
            descs[k].wait()
            pltpu.sync_copy(bufs[k], acc_sh.at[idx_v.at[j]], add=True)
        plsc.subcore_barrier()
        pltpu.sync_copy(acc_sh.at[pl.ds(sid * stripe, stripe)],
                        out_hbm.at[pl.ds(cid * _N + sid * stripe, stripe)])

    return sk(msg, idx2d, zeros)


def _prep_params(params):
    f32 = jnp.float32
    pairs = ('00', '01', '10', '11')

    w1c = jnp.concatenate(
        [jnp.pad(params[p]['w1'], ((0, 15), (0, 0))) for p in pairs], axis=1)
    b1c = jnp.concatenate([params[p]['b1'] for p in pairs]).reshape(1, 128)
    g1c = jnp.concatenate([params[p]['g1'] for p in pairs]).reshape(1, 128)
    be1c = jnp.concatenate([params[p]['be1'] for p in pairs]).reshape(1, 128)
    b2c = jnp.concatenate([params[p]['b2'] for p in pairs]).reshape(1, 128)
    g2c = jnp.concatenate([params[p]['g2'] for p in pairs]).reshape(1, 128)
    be2c = jnp.concatenate([params[p]['be2'] for p in pairs]).reshape(1, 128)

    w2bd = jnp.zeros((128, 128), f32)
    for k, p in enumerate(pairs):
        w2bd = w2bd.at[32 * k:32 * k + 32, 32 * k:32 * k + 32].set(
            params[p]['w2'])

    mg = jnp.kron(jnp.eye(4, dtype=f32), jnp.full((32, 32), 1.0 / 32.0, f32))

    w3_11 = params['11']['w3'].reshape(32, 16, 16, 3).transpose(0, 3, 1, 2)
    w3_11 = w3_11.reshape(32, 768)
    b3_11 = params['11']['b3'].reshape(16, 16, 3).transpose(2, 0, 1).reshape(768)

    w3bd = jnp.zeros((128, 1536), f32)
    offs = (0, 256, 512, 768)
    mats = (params['00']['w3'], params['01']['w3'], params['10']['w3'], w3_11)
    for k in range(4):
        w3bd = w3bd.at[32 * k:32 * k + 32,
                       offs[k]:offs[k] + mats[k].shape[1]].set(mats[k])
    b3c = jnp.concatenate([params['00']['b3'], params['01']['b3'],
                           params['10']['b3'], b3_11]).reshape(1, 1536)

    t16 = jnp.tile(jnp.eye(16, dtype=f32), (1, 16))
    s256 = jnp.kron(jnp.eye(16, dtype=f32), jnp.ones((16, 1), f32))

    sel_np = np.zeros((34, 4, 224), np.float32)
    for blk, c in ((0, 0), (2, 4), (3, 5), (4, 6)):
        sel_np[c, 0, blk * 16:blk * 16 + 16] = 1.0
    for mo in range(3):
        sel_np[1 + mo, 1 + mo, 16:32] = 1.0
        for f in range(3):
            for mi in range(3):
                blk = 5 + f * 3 + mi
                sel_np[7 + mo * 9 + mi * 3 + f, 1 + mo,
                       blk * 16:blk * 16 + 16] = 1.0
    sel = jnp.asarray(sel_np.reshape(34, 896))
    s224 = jnp.asarray(np.kron(np.ones((14, 1), np.float32),
                               np.eye(16, dtype=np.float32)))

    pp = (w1c, b1c, g1c, be1c, w2bd, b2c, g2c, be2c, mg, w3bd, b3c, t16,
          s256, sel, s224)

    s0t = params['self']['0'][0].T
    k1 = jnp.kron(jnp.eye(3, dtype=f32), params['self']['1'][0].T)
    return pp, s0t, k1


def _prep_edges(h0, h1, w, r, basis_00, basis_01, basis_10, basis_11):
    f32 = jnp.float32
    hcat = jnp.concatenate(
        [h0[:, :, 0], h1[:, :, 0], h1[:, :, 1], h1[:, :, 2]], axis=1)
    feat = jnp.concatenate(
        [w, r, jnp.zeros((_E, 15), f32)], axis=1)
    bb = jnp.concatenate(
        [basis_00.reshape(_E, 1), basis_01.reshape(_E, 3),
         basis_10.reshape(_E, 3), basis_11.reshape(_E, 27)], axis=1)
    return hcat, feat, bb


def kernel(h0, h1, edge_index, w, r, basis_00, basis_01, basis_10, basis_11,
           params):
    hcat, feat, bb = _prep_edges(h0, h1, w, r, basis_00, basis_01,
                                 basis_10, basis_11)
    pp, s0t, k1 = _prep_params(params)

    src2d = edge_index[0].reshape(_E // 128, 128)
    dst2d = edge_index[1].reshape(_E // 128, 128)

    hpad = jnp.pad(hcat, ((0, 0), (0, 64)))
    hs = _sc_gather(hpad, src2d)
    msg = _run_edge(feat, hs, bb, pp)
    part = _sc_scatter(msg, dst2d, jnp.zeros((_N, 128), jnp.float32))
    o0, o1 = _run_combine(part, hcat, s0t, k1)

    out0 = o0.reshape(_N, _M, 1)
    out1 = o1.reshape(_N, 3, _M).transpose(0, 2, 1)
    return out0, out1


# speedup vs baseline: 1.0641x; 1.0641x over previous
import functools

import jax
import jax.numpy as jnp
import numpy as np
from jax import lax
from jax.experimental import pallas as pl
from jax.experimental.pallas import tpu as pltpu
from jax.experimental.pallas import tpu_sc as plsc

_N = 4096
_E = 32768
_M = 16

_NC = 2
_NS = 16
_NW = _NC * _NS

_EDGE_BLOCK = 512
_NODE_BLOCK = 1024


def _edge_body(feat_ref, hs_ref, bb_ref,
               w1_ref, b1_ref, g1_ref, be1_ref,
               w2_ref, b2_ref, g2_ref, be2_ref, mg_ref, w3_ref, b3_ref,
               t16_ref, s256_ref, sel_ref, s224_ref, out_ref):
    f32 = jnp.float32

    def mm(a, b):
        return jnp.dot(a, b, preferred_element_type=f32)

    mg = mg_ref[...]

    def ln_relu(y, g, be):
        mu = mm(y, mg)
        var = mm(y * y, mg) - mu * mu
        yn = (y - mu) / jnp.sqrt(var + 1e-5) * g + be
        return jnp.maximum(yn, 0.0)

    y = mm(feat_ref[...], w1_ref[...]) + b1_ref[...]
    y = ln_relu(y, g1_ref[...], be1_ref[...])
    y = mm(y, w2_ref[...]) + b2_ref[...]
    z = ln_relu(y, g2_ref[...], be2_ref[...])
    r = mm(z, w3_ref[...]) + b3_ref[...]

    hs = hs_ref[...]
    t16 = t16_ref[...]
    s256 = s256_ref[...]

    ht = [mm(hs[:, 16 * j:16 * j + 16], t16) for j in range(4)]
    rs = [r[:, 256 * k:256 * k + 256] for k in range(6)]

    prods = [rs[0] * ht[0], rs[1] * ht[0]]
    prods += [rs[2] * ht[1 + mi] for mi in range(3)]
    for f in range(3):
        prods += [rs[3 + f] * ht[1 + mi] for mi in range(3)]
    qall = jnp.concatenate([mm(p, s256) for p in prods], axis=1)

    bt = mm(bb_ref[...], sel_ref[...])
    s224 = s224_ref[...]
    parts = [mm(qall * bt[:, 224 * j:224 * j + 224], s224) for j in range(4)]
    parts.append(jnp.ones((qall.shape[0], 16), f32))
    parts.append(jnp.zeros((qall.shape[0], 48), f32))
    out_ref[...] = jnp.concatenate(parts, axis=1)


def _run_edge(feat, hs, bb, pp, interpret=False):
    bsz = _EDGE_BLOCK
    grid = (_E // bsz,)

    def blk(shape):
        return pl.BlockSpec(shape, lambda i: (i, 0))

    def full(a):
        return pl.BlockSpec(a.shape, lambda i: (0, 0))

    in_specs = [blk((bsz, 32)), blk((bsz, 128)), blk((bsz, 34))]
    in_specs += [full(p) for p in pp]
    return pl.pallas_call(
        _edge_body,
        grid=grid,
        in_specs=in_specs,
        out_specs=blk((bsz, 128)),
        out_shape=jax.ShapeDtypeStruct((_E, 128), jnp.float32),
        interpret=interpret,
    )(feat, hs, bb, *pp)


def _combine_body(p0_ref, p1_ref, hc_ref, s0t_ref, k1_ref, o0_ref, o1_ref):
    f32 = jnp.float32
    acc = p0_ref[...] + p1_ref[...]
    cnt = acc[:, 64:65]
    denom = jnp.maximum(cnt, 1.0)
    hc = hc_ref[...]
    s0 = jnp.dot(hc[:, 0:16], s0t_ref[...], preferred_element_type=f32)
    s1 = jnp.dot(hc[:, 16:64], k1_ref[...], preferred_element_type=f32)
    o0_ref[...] = (acc[:, 0:16] + cnt * s0) / denom
    o1_ref[...] = (acc[:, 16:64] + cnt * s1) / denom


def _run_combine(part, hcat, s0t, k1, interpret=False):
    bsz = _NODE_BLOCK
    grid = (_N // bsz,)

    def blk(shape):
        return pl.BlockSpec(shape, lambda i: (i, 0))

    def full(a):
        return pl.BlockSpec(a.shape, lambda i: (0, 0))

    return pl.pallas_call(
        _combine_body,
        grid=grid,
        in_specs=[blk((bsz, 128)), blk((bsz, 128)), blk((bsz, 64)),
                  full(s0t), full(k1)],
        out_specs=[blk((bsz, 16)), blk((bsz, 48))],
        out_shape=[jax.ShapeDtypeStruct((_N, 16), jnp.float32),
                   jax.ShapeDtypeStruct((_N, 48), jnp.float32)],
        interpret=interpret,
    )(part[:_N], part[_N:], hcat, s0t, k1)


def _sc_mesh():
    return plsc.VectorSubcoreMesh(core_axis_name="c", subcore_axis_name="s",
                                  num_cores=_NC, num_subcores=_NS)


def _sc_gather(table, idx2d):
    rows_per_w = _E // _NW
    chunks = rows_per_w // 128

    @functools.partial(
        pl.kernel,
        out_type=jax.ShapeDtypeStruct((_E, 128), jnp.float32),
        mesh=_sc_mesh(),
        scratch_types=[pltpu.VMEM((chunks, 128), jnp.int32),
                       pltpu.VMEM((128, 128), jnp.float32),
                       pltpu.VMEM((128, 128), jnp.float32),
                       pltpu.SemaphoreType.DMA,
                       pltpu.SemaphoreType.DMA],
    )
    def gk(table_hbm, idx_hbm, out_hbm, idx_v, row0, row1, sem0, sem1):
        wid = lax.axis_index("s") * _NC + lax.axis_index("c")
        base = wid * rows_per_w
        pltpu.sync_copy(idx_hbm.at[pl.ds(wid * chunks, chunks)], idx_v)
        bufs = (row0, row1)
        sems = (sem0, sem1)
        descs = [None, None]
        for j in range(chunks):
            k = j % 2
            if descs[k] is not None:
                descs[k].wait()
                pltpu.sync_copy(bufs[k], out_hbm.at[pl.ds(base + (j - 2) * 128,
                                                          128)])
            descs[k] = pltpu.async_copy(table_hbm.at[idx_v.at[j]], bufs[k],
                                        sems[k])
        for j in range(chunks - 2, chunks):
            k = j % 2
            descs[k].wait()
            pltpu.sync_copy(bufs[k], out_hbm.at[pl.ds(base + j * 128, 128)])

    return gk(table, idx2d)


def _sc_scatter(msg, idx2d, zeros):
    rows_per_w = _E // _NW
    chunks = rows_per_w // 128
    stripe = _N // _NS

    @functools.partial(
        pl.kernel,
        out_type=jax.ShapeDtypeStruct((2 * _N, 128), jnp.float32),
        mesh=_sc_mesh(),
        scratch_types=[pltpu.VMEM((chunks, 128), jnp.int32),
                       pltpu.VMEM((128, 128), jnp.float32),
                       pltpu.VMEM((128, 128), jnp.float32),
                       pltpu.VMEM_SHARED((_N, 128), jnp.float32),
                       pltpu.SemaphoreType.DMA,
                       pltpu.SemaphoreType.DMA],
    )
    def sk(msg_hbm, idx_hbm, z_hbm, out_hbm, idx_v, buf0, buf1, acc_sh,
           sem0, sem1):
        cid = lax.axis_index("c")
        sid = lax.axis_index("s")
        wid = sid * _NC + cid
        base = wid * rows_per_w
        pltpu.sync_copy(z_hbm.at[pl.ds(sid * stripe, stripe)],
                        acc_sh.at[pl.ds(sid * stripe, stripe)])
        plsc.subcore_barrier()
        pltpu.sync_copy(idx_hbm.at[pl.ds(wid * chunks, chunks)], idx_v)
        bufs = (buf0, buf1)
        sems = (sem0, sem1)
        descs = [None, None]
        for j in range(chunks):
            k = j % 2
            if descs[k] is not None:
                descs[k].wait()
                pltpu.sync_copy(bufs[k], acc_sh.at[idx_v.at[j - 2]], add=True)
            descs[k] = pltpu.async_copy(
                msg_hbm.at[pl.ds(base + j * 128, 128)], bufs[k], sems[k])
        for j in range(chunks - 2, chunks):
            k = j % 2
            descs[k].wait()
            pltpu.sync_copy(bufs[k], acc_sh.at[idx_v.at[j]], add=True)
        plsc.subcore_barrier()
        pltpu.sync_copy(acc_sh.at[pl.ds(sid * stripe, stripe)],
                        out_hbm.at[pl.ds(cid * _N + sid * stripe, stripe)])

    return sk(msg, idx2d, zeros)


def _prep_params(params):
    f32 = jnp.float32
    pairs = ('00', '01', '10', '11')

    w1c = jnp.concatenate(
        [jnp.pad(params[p]['w1'], ((0, 15), (0, 0))) for p in pairs], axis=1)
    b1c = jnp.concatenate([params[p]['b1'] for p in pairs]).reshape(1, 128)
    g1c = jnp.concatenate([params[p]['g1'] for p in pairs]).reshape(1, 128)
    be1c = jnp.concatenate([params[p]['be1'] for p in pairs]).reshape(1, 128)
    b2c = jnp.concatenate([params[p]['b2'] for p in pairs]).reshape(1, 128)
    g2c = jnp.concatenate([params[p]['g2'] for p in pairs]).reshape(1, 128)
    be2c = jnp.concatenate([params[p]['be2'] for p in pairs]).reshape(1, 128)

    w2bd = jnp.zeros((128, 128), f32)
    for k, p in enumerate(pairs):
        w2bd = w2bd.at[32 * k:32 * k + 32, 32 * k:32 * k + 32].set(
            params[p]['w2'])

    mg = jnp.kron(jnp.eye(4, dtype=f32), jnp.full((32, 32), 1.0 / 32.0, f32))

    w3_11 = params['11']['w3'].reshape(32, 16, 16, 3).transpose(0, 3, 1, 2)
    w3_11 = w3_11.reshape(32, 768)
    b3_11 = params['11']['b3'].reshape(16, 16, 3).transpose(2, 0, 1).reshape(768)

    w3bd = jnp.zeros((128, 1536), f32)
    offs = (0, 256, 512, 768)
    mats = (params['00']['w3'], params['01']['w3'], params['10']['w3'], w3_11)
    for k in range(4):
        w3bd = w3bd.at[32 * k:32 * k + 32,
                       offs[k]:offs[k] + mats[k].shape[1]].set(mats[k])
    b3c = jnp.concatenate([params['00']['b3'], params['01']['b3'],
                           params['10']['b3'], b3_11]).reshape(1, 1536)

    t16 = jnp.tile(jnp.eye(16, dtype=f32), (1, 16))
    s256 = jnp.kron(jnp.eye(16, dtype=f32), jnp.ones((16, 1), f32))

    sel_np = np.zeros((34, 4, 224), np.float32)
    for blk, c in ((0, 0), (2, 4), (3, 5), (4, 6)):
        sel_np[c, 0, blk * 16:blk * 16 + 16] = 1.0
    for mo in range(3):
        sel_np[1 + mo, 1 + mo, 16:32] = 1.0
        for f in range(3):
            for mi in range(3):
                blk = 5 + f * 3 + mi
                sel_np[7 + f * 9 + mo * 3 + mi, 1 + mo,
                       blk * 16:blk * 16 + 16] = 1.0
    sel = jnp.asarray(sel_np.reshape(34, 896))
    s224 = jnp.asarray(np.kron(np.ones((14, 1), np.float32),
                               np.eye(16, dtype=np.float32)))

    pp = (w1c, b1c, g1c, be1c, w2bd, b2c, g2c, be2c, mg, w3bd, b3c, t16,
          s256, sel, s224)

    s0t = params['self']['0'][0].T
    k1 = jnp.kron(jnp.eye(3, dtype=f32), params['self']['1'][0].T)
    return pp, s0t, k1


def _prep_edges(h0, h1, w, r, basis_00, basis_01, basis_10, basis_11):
    f32 = jnp.float32
    hcat = jnp.concatenate(
        [h0[:, :, 0], h1[:, :, 0], h1[:, :, 1], h1[:, :, 2]], axis=1)
    feat = jnp.concatenate(
        [w, r, jnp.zeros((_E, 15), f32)], axis=1)
    b11 = basis_11.reshape(_E, 3, 3, 3).transpose(0, 3, 1, 2).reshape(_E, 27)
    bb = jnp.concatenate(
        [basis_00.reshape(_E, 1), basis_01.reshape(_E, 3),
         basis_10.reshape(_E, 3), b11], axis=1)
    return hcat, feat, bb


def kernel(h0, h1, edge_index, w, r, basis_00, basis_01, basis_10, basis_11,
           params):
    hcat, feat, bb = _prep_edges(h0, h1, w, r, basis_00, basis_01,
                                 basis_10, basis_11)
    pp, s0t, k1 = _prep_params(params)

    src2d = edge_index[0].reshape(_E // 128, 128)
    dst2d = edge_index[1].reshape(_E // 128, 128)

    hpad = jnp.pad(hcat, ((0, 0), (0, 64)))
    hs = _sc_gather(hpad, src2d)
    msg = _run_edge(feat, hs, bb, pp)
    part = _sc_scatter(msg, dst2d, jnp.zeros((_N, 128), jnp.float32))
    o0, o1 = _run_combine(part, hcat, s0t, k1)

    out0 = o0.reshape(_N, _M, 1)
    out1 = o1.reshape(_N, 3, _M).transpose(0, 2, 1)
    return out0, out1
